# Initial kernel scaffold; baseline (speedup 1.0000x reference)
import jax
import jax.numpy as jnp
from jax.experimental import pallas as pl


def kernel(edge_attr, W_type, W_dir):
    raise NotImplementedError("write your pallas kernel here")


# speedup vs baseline: 4.3936x; 4.3936x over previous
import functools

import jax
import jax.numpy as jnp
from jax import lax
from jax.experimental import pallas as pl
from jax.experimental.pallas import tpu as pltpu
from jax.experimental.pallas import tpu_sc as plsc

EMB = 16
NC = 2
NS = 16
NW = NC * NS


def _edge_encode(n_edges):
    per_w = n_edges // NW
    chunk = 2000
    while per_w % chunk:
        chunk -= 16
    n_chunks = per_w // chunk

    mesh = plsc.VectorSubcoreMesh(core_axis_name="c", subcore_axis_name="s")

    @functools.partial(
        pl.kernel,
        mesh=mesh,
        out_type=jax.ShapeDtypeStruct((n_edges * EMB,), jnp.float32),
        compiler_params=pltpu.CompilerParams(needs_layout_passes=False),
        scratch_types=[
            pltpu.VMEM((chunk * 2,), jnp.int32),
            pltpu.VMEM((chunk * EMB,), jnp.float32),
            pltpu.VMEM((2 * EMB,), jnp.float32),
            pltpu.VMEM((2 * EMB,), jnp.float32),
            pltpu.VMEM((4 * EMB,), jnp.float32),
        ],
    )
    def run(attr_hbm, wt_hbm, wd_hbm, out_hbm, attr_v, rows_v, wt_v, wd_v, lut_v):
        wid = lax.axis_index("s") * NC + lax.axis_index("c")
        iota = lax.iota(jnp.int32, 16)

        pltpu.sync_copy(wt_hbm, wt_v)
        pltpu.sync_copy(wd_hbm, wd_v)
        wt0 = wt_v[pl.ds(0, 16)]
        wt1 = wt_v[pl.ds(16, 16)]
        wd0 = wd_v[pl.ds(0, 16)]
        wd1 = wd_v[pl.ds(16, 16)]
        lut_v[pl.ds(0, 16)] = wt0 + wd0
        lut_v[pl.ds(16, 16)] = wt0 + wd1
        lut_v[pl.ds(32, 16)] = wt1 + wd0
        lut_v[pl.ds(48, 16)] = wt1 + wd1

        def chunk_body(ci, carry):
            base = wid * per_w + ci * chunk
            pltpu.sync_copy(attr_hbm.at[pl.ds(base * 2, chunk * 2)], attr_v)

            def group_body(g, c2):
                pairs = g * 32 + iota * 2
                a0 = plsc.load_gather(attr_v, [pairs])
                a1 = plsc.load_gather(attr_v, [pairs + 1])
                cb = (a0 * 2 + a1) * 16
                pos = g * 256 + iota * 16
                for d in range(16):
                    pm = jnp.bitwise_and(iota + d, 15)
                    val = plsc.load_gather(lut_v, [cb + pm])
                    plsc.store_scatter(rows_v, [pos + pm], val)
                return c2

            lax.fori_loop(0, chunk // 16, group_body, 0)
            pltpu.sync_copy(rows_v, out_hbm.at[pl.ds(base * EMB, chunk * EMB)])
            return carry

        lax.fori_loop(0, n_chunks, chunk_body, 0)

    return run


def kernel(edge_attr, W_type, W_dir):
    n_edges = edge_attr.shape[0]
    run = _edge_encode(n_edges)
    out = run(
        edge_attr.astype(jnp.int32).reshape(n_edges * 2),
        W_type.reshape(2 * EMB),
        W_dir.reshape(2 * EMB),
    )
    return out.reshape(n_edges, EMB)
